# initial kernel scaffold (unmeasured)
import jax
import jax.numpy as jnp
from jax import lax
from jax.experimental import pallas as pl
from jax.experimental.pallas import tpu as pltpu

_DeviceIdType = getattr(pl, "DeviceIdType", None) or pltpu.DeviceIdType
_sem_signal = getattr(pl, "semaphore_signal", None) or pltpu.semaphore_signal
_sem_wait = getattr(pl, "semaphore_wait", None) or pltpu.semaphore_wait
_CompilerParams = getattr(pltpu, "CompilerParams", None) or getattr(
    pltpu, "TPUCompilerParams"
)


def kernel(x, dest):
    n_per, d = x.shape
    dest2 = dest.reshape(1, n_per)

    def body(x_ref, d_ref, out_ref, comm_x, comm_d, send_sems, recv_sems):
        my_x = lax.axis_index("x")
        my_y = lax.axis_index("y")
        my_z = lax.axis_index("z")
        peer = (1 - my_x, my_y, my_z)

        barrier_sem = pltpu.get_barrier_semaphore()
        _sem_signal(
            barrier_sem, inc=1, device_id=peer, device_id_type=_DeviceIdType.MESH
        )
        _sem_wait(barrier_sem, 1)

        comm_x[0, :, :] = x_ref[:, :].astype(jnp.bfloat16)
        comm_d[0, :] = d_ref[0, :]

        rdma_x = pltpu.make_async_remote_copy(
            src_ref=comm_x.at[0],
            dst_ref=comm_x.at[1],
            send_sem=send_sems.at[0],
            recv_sem=recv_sems.at[0],
            device_id=peer,
            device_id_type=_DeviceIdType.MESH,
        )
        rdma_d = pltpu.make_async_remote_copy(
            src_ref=comm_d.at[0],
            dst_ref=comm_d.at[1],
            send_sem=send_sems.at[1],
            recv_sem=recv_sems.at[1],
            device_id=peer,
            device_id_type=_DeviceIdType.MESH,
        )
        rdma_x.start()
        rdma_d.start()
        rdma_d.wait()
        rdma_x.wait()

        d_all = comm_d[:, :]
        maskf = (d_all == my_x).astype(jnp.float32)
        ii = lax.broadcasted_iota(jnp.int32, (n_per, n_per), 0)
        jj = lax.broadcasted_iota(jnp.int32, (n_per, n_per), 1)
        tri = (ii <= jj).astype(jnp.float32)
        cum = lax.dot_general(
            maskf, tri, (((1,), (0,)), ((), ())),
            preferred_element_type=jnp.float32,
        )
        tot = lax.slice(cum, (0, n_per - 1), (2, n_per))
        t_mine = lax.slice(tot, (0, 0), (1, 1))
        t_peer = lax.slice(tot, (1, 0), (2, 1))
        off0 = jnp.where(my_x == 0, 0.0, t_peer)
        off1 = jnp.where(my_x == 0, t_mine, 0.0)
        m0 = lax.slice(maskf, (0, 0), (1, n_per))
        m1 = lax.slice(maskf, (1, 0), (2, n_per))
        c0 = lax.slice(cum, (0, 0), (1, n_per))
        c1 = lax.slice(cum, (1, 0), (2, n_per))
        pos0 = jnp.where(m0 > 0, c0 - 1.0 + off0, -1.0)
        pos1 = jnp.where(m1 > 0, c1 - 1.0 + off1, -1.0)
        rowi = lax.broadcasted_iota(jnp.float32, (n_per, n_per), 0)
        p0 = (pos0 == rowi).astype(jnp.bfloat16)
        p1 = (pos1 == rowi).astype(jnp.bfloat16)
        acc = lax.dot_general(
            p0, comm_x[0], (((1,), (0,)), ((), ())),
            preferred_element_type=jnp.float32,
        )
        acc = acc + lax.dot_general(
            p1, comm_x[1], (((1,), (0,)), ((), ())),
            preferred_element_type=jnp.float32,
        )
        out_ref[:, :] = acc

    return pl.pallas_call(
        body,
        out_shape=jax.ShapeDtypeStruct((n_per, d), jnp.float32),
        in_specs=[
            pl.BlockSpec(memory_space=pltpu.VMEM),
            pl.BlockSpec(memory_space=pltpu.VMEM),
        ],
        out_specs=pl.BlockSpec(memory_space=pltpu.VMEM),
        scratch_shapes=[
            pltpu.VMEM((2, n_per, d), jnp.bfloat16),
            pltpu.VMEM((2, n_per), jnp.int32),
            pltpu.SemaphoreType.DMA((2,)),
            pltpu.SemaphoreType.DMA((2,)),
        ],
        compiler_params=_CompilerParams(collective_id=0),
    )(x, dest2)


# baseline (device time: 21330 ns/iter reference)
import jax
import jax.numpy as jnp
from jax import lax
from jax.experimental import pallas as pl
from jax.experimental.pallas import tpu as pltpu

_DeviceIdType = getattr(pl, "DeviceIdType", None) or pltpu.DeviceIdType
_sem_signal = getattr(pl, "semaphore_signal", None) or pltpu.semaphore_signal
_sem_wait = getattr(pl, "semaphore_wait", None) or pltpu.semaphore_wait
_CompilerParams = getattr(pltpu, "CompilerParams", None) or getattr(
    pltpu, "TPUCompilerParams"
)


def kernel(x, dest):
    n_per, d = x.shape
    dest2 = dest.reshape(1, n_per)

    def body(x_ref, d_ref, out_ref, comm_x, comm_d, send_sems, recv_sems):
        my_x = lax.axis_index("x")
        my_y = lax.axis_index("y")
        my_z = lax.axis_index("z")
        peer = (1 - my_x, my_y, my_z)

        barrier_sem = pltpu.get_barrier_semaphore()
        _sem_signal(
            barrier_sem, inc=1, device_id=peer, device_id_type=_DeviceIdType.MESH
        )
        _sem_wait(barrier_sem, 1)

        comm_x[0, :, :] = x_ref[:, :].astype(jnp.bfloat16)
        comm_d[0, :] = d_ref[0, :]

        rdma_x = pltpu.make_async_remote_copy(
            src_ref=comm_x.at[0],
            dst_ref=comm_x.at[1],
            send_sem=send_sems.at[0],
            recv_sem=recv_sems.at[0],
            device_id=peer,
            device_id_type=_DeviceIdType.MESH,
        )
        rdma_d = pltpu.make_async_remote_copy(
            src_ref=comm_d.at[0],
            dst_ref=comm_d.at[1],
            send_sem=send_sems.at[1],
            recv_sem=recv_sems.at[1],
            device_id=peer,
            device_id_type=_DeviceIdType.MESH,
        )
        rdma_x.start()
        rdma_d.start()
        rdma_d.wait()
        rdma_x.wait()

        d_all = comm_d[:, :]
        maskf = (d_all == my_x).astype(jnp.float32)
        ii = lax.broadcasted_iota(jnp.int32, (n_per, n_per), 0)
        jj = lax.broadcasted_iota(jnp.int32, (n_per, n_per), 1)
        tri = (ii <= jj).astype(jnp.float32)
        cum = lax.dot_general(
            maskf, tri, (((1,), (0,)), ((), ())),
            preferred_element_type=jnp.float32,
        )
        tot = lax.slice(cum, (0, n_per - 1), (2, n_per))
        t_mine = lax.slice(tot, (0, 0), (1, 1))
        t_peer = lax.slice(tot, (1, 0), (2, 1))
        off0 = jnp.where(my_x == 0, 0.0, t_peer)
        off1 = jnp.where(my_x == 0, t_mine, 0.0)
        m0 = lax.slice(maskf, (0, 0), (1, n_per))
        m1 = lax.slice(maskf, (1, 0), (2, n_per))
        c0 = lax.slice(cum, (0, 0), (1, n_per))
        c1 = lax.slice(cum, (1, 0), (2, n_per))
        pos0 = jnp.where(m0 > 0, c0 - 1.0 + off0, -1.0)
        pos1 = jnp.where(m1 > 0, c1 - 1.0 + off1, -1.0)
        rowi = lax.broadcasted_iota(jnp.int32, (n_per, n_per), 0).astype(
            jnp.float32
        )
        p0 = (pos0 == rowi).astype(jnp.bfloat16)
        p1 = (pos1 == rowi).astype(jnp.bfloat16)
        acc = lax.dot_general(
            p0, comm_x[0], (((1,), (0,)), ((), ())),
            preferred_element_type=jnp.float32,
        )
        acc = acc + lax.dot_general(
            p1, comm_x[1], (((1,), (0,)), ((), ())),
            preferred_element_type=jnp.float32,
        )
        out_ref[:, :] = acc

    return pl.pallas_call(
        body,
        out_shape=jax.ShapeDtypeStruct((n_per, d), jnp.float32),
        in_specs=[
            pl.BlockSpec(memory_space=pltpu.VMEM),
            pl.BlockSpec(memory_space=pltpu.VMEM),
        ],
        out_specs=pl.BlockSpec(memory_space=pltpu.VMEM),
        scratch_shapes=[
            pltpu.VMEM((2, n_per, d), jnp.bfloat16),
            pltpu.VMEM((2, n_per), jnp.int32),
            pltpu.SemaphoreType.DMA((2,)),
            pltpu.SemaphoreType.DMA((2,)),
        ],
        compiler_params=_CompilerParams(collective_id=0),
    )(x, dest2)
